# baseline (speedup 1.0000x reference)
import functools

import jax
import jax.numpy as jnp
from jax import lax
from jax.experimental import pallas as pl
from jax.experimental.pallas import tpu as pltpu
from jax.experimental.pallas import tpu_sc as plsc

VOCAB = 1000
VPAD = 1024
BT = 51200
NSPLIT = 2
HBT = BT // NSPLIT
NC, NS = 2, 16
NW = NC * NS
PER_TILE = HBT // NW
CHUNK = 40
NCHUNK = PER_TILE // CHUNK
DEPAD_BLOCK = 256
LGRP = 128


def _row_logsumexp(table):
    def body(t_ref, o_ref):
        x = t_ref[...]
        m = jnp.max(x, axis=1, keepdims=True)
        s = jnp.sum(jnp.exp(x - m), axis=1, keepdims=True)
        o_ref[...] = jnp.log(s) + m

    return pl.pallas_call(
        body,
        out_shape=jax.ShapeDtypeStruct((VOCAB, 1), jnp.float32),
    )(table)


def _sc_gather_and_loss(idx_half, idx2_half, table_pad, table_flat, lse):
    mesh = plsc.VectorSubcoreMesh(core_axis_name="c", subcore_axis_name="s")

    @functools.partial(
        pl.kernel,
        mesh=mesh,
        out_type=[
            jax.ShapeDtypeStruct((HBT, VPAD), jnp.float32),
            jax.ShapeDtypeStruct((NW, 16), jnp.float32),
        ],
        scratch_types=[
            pltpu.VMEM((PER_TILE,), jnp.int32),
            pltpu.VMEM((PER_TILE,), jnp.int32),
            pltpu.VMEM((CHUNK, VPAD), jnp.float32),
            pltpu.VMEM((CHUNK, VPAD), jnp.float32),
            pltpu.VMEM((PER_TILE,), jnp.float32),
            pltpu.VMEM((PER_TILE,), jnp.float32),
            pltpu.VMEM((16,), jnp.float32),
            pltpu.SemaphoreType.DMA,
            pltpu.SemaphoreType.DMA,
            pltpu.SemaphoreType.DMA,
            pltpu.SemaphoreType.DMA,
            pltpu.SemaphoreType.DMA,
        ],
    )
    def k(idx_hbm, idx2_hbm, table_hbm, tflat_hbm, lse_hbm,
          out_hbm, part_hbm,
          idx_v, idx2_v, rows0_v, rows1_v, tl_v, ls_v, acc_v,
          sem_g0, sem_g1, sem_s0, sem_s1, sem_e):
        rows = (rows0_v, rows1_v)
        sem_g = (sem_g0, sem_g1)
        sem_s = (sem_s0, sem_s1)
        wid = lax.axis_index("s") * NC + lax.axis_index("c")
        base = wid * PER_TILE
        pltpu.sync_copy(idx_hbm.at[pl.ds(base, PER_TILE)], idx_v)
        pltpu.sync_copy(idx2_hbm.at[pl.ds(base, PER_TILE)], idx2_v)

        def gather_rows(c, b, start):
            cp = pltpu.make_async_copy(
                table_hbm.at[idx_v.at[pl.ds(c * CHUNK, CHUNK)]],
                rows[b], sem_g[b],
            )
            if start:
                cp.start()
            return cp

        gather_rows(0, 0, True)
        gather_rows(1, 1, True)

        elem_cps = []
        for og in range(0, PER_TILE, LGRP):
            n = min(LGRP, PER_TILE - og)
            elem_cps.append(pltpu.async_copy(
                tflat_hbm.at[idx2_v.at[pl.ds(og, n)]],
                tl_v.at[pl.ds(og, n)], sem_e,
            ))
            elem_cps.append(pltpu.async_copy(
                lse_hbm.at[idx_v.at[pl.ds(og, n)]],
                ls_v.at[pl.ds(og, n)], sem_e,
            ))

        def pair_body(p, carry):
            for b in range(2):
                c = 2 * p + b
                o = c * CHUNK
                gather_rows(c, b, False).wait()
                scat = pltpu.async_copy(
                    rows[b],
                    out_hbm.at[pl.ds(base + o, CHUNK)],
                    sem_s[b],
                )
                scat.wait()
                @pl.when(c + 2 < NCHUNK)
                def _():
                    gather_rows(c + 2, b, True)
            return carry

        lax.fori_loop(0, NCHUNK // 2, pair_body, 0)

        for cp in elem_cps:
            cp.wait()

        def loss_body(g, acc):
            og = g * 16
            return acc + (ls_v[pl.ds(og, 16)] - tl_v[pl.ds(og, 16)])

        acc = lax.fori_loop(
            0, PER_TILE // 16, loss_body, jnp.zeros((16,), jnp.float32)
        )
        acc_v[...] = acc * (1.0 / BT)
        pltpu.sync_copy(acc_v, part_hbm.at[wid])

    return k(idx_half, idx2_half, table_pad, table_flat, lse)


def _depad_first(padded):
    def body(t_ref, o_ref):
        o_ref[...] = t_ref[:, :VOCAB]

    return pl.pallas_call(
        body,
        grid=(HBT // DEPAD_BLOCK,),
        in_specs=[pl.BlockSpec((DEPAD_BLOCK, VPAD), lambda g: (g, 0))],
        out_specs=pl.BlockSpec((DEPAD_BLOCK, VOCAB), lambda g: (g, 0)),
        out_shape=jax.ShapeDtypeStruct((BT, VOCAB), jnp.float32),
    )(padded)


def _depad_second(prev, padded):
    def body(prev_ref, t_ref, o_ref):
        o_ref[...] = t_ref[:, :VOCAB]

    return pl.pallas_call(
        body,
        grid=(HBT // DEPAD_BLOCK,),
        in_specs=[
            pl.BlockSpec(memory_space=pl.ANY),
            pl.BlockSpec((DEPAD_BLOCK, VPAD), lambda g: (g, 0)),
        ],
        out_specs=pl.BlockSpec(
            (DEPAD_BLOCK, VOCAB), lambda g: (g + HBT // DEPAD_BLOCK, 0)
        ),
        out_shape=jax.ShapeDtypeStruct((BT, VOCAB), jnp.float32),
        input_output_aliases={0: 0},
    )(prev, padded)


def kernel(index, targets, token_embedding_table):
    idx_flat = jnp.transpose(index).reshape(-1)
    tgt_flat = targets.reshape(-1)
    idx2_flat = idx_flat * VOCAB + tgt_flat
    lse = _row_logsumexp(token_embedding_table).reshape(VOCAB)
    table_pad = jnp.pad(token_embedding_table, ((0, 0), (0, VPAD - VOCAB)))
    table_flat = token_embedding_table.reshape(-1)

    pads, parts = [], []
    for h in range(NSPLIT):
        sl = slice(h * HBT, (h + 1) * HBT)
        p, pt = _sc_gather_and_loss(
            idx_flat[sl], idx2_flat[sl], table_pad, table_flat, lse
        )
        pads.append(p)
        parts.append(pt)

    logits = _depad_first(pads[0])
    logits = _depad_second(logits, pads[1])
    loss = jnp.sum(jnp.stack(parts))
    return (logits, loss)


# speedup vs baseline: 1.3544x; 1.3544x over previous
import functools

import jax
import jax.numpy as jnp
from jax import lax
from jax.experimental import pallas as pl
from jax.experimental.pallas import tpu as pltpu
from jax.experimental.pallas import tpu_sc as plsc

VOCAB = 1000
VPAD = 1024
BT = 51200
NC, NS = 2, 16
NW = NC * NS
PER_TILE = BT // NW
CHUNK = 16
NCHUNK = PER_TILE // CHUNK
LGRP = 128


def _row_logsumexp(table):
    def body(t_ref, o_ref):
        x = t_ref[...]
        m = jnp.max(x, axis=1, keepdims=True)
        s = jnp.sum(jnp.exp(x - m), axis=1, keepdims=True)
        o_ref[...] = jnp.log(s) + m

    return pl.pallas_call(
        body,
        out_shape=jax.ShapeDtypeStruct((VOCAB, 1), jnp.float32),
    )(table)


def _sc_gather_and_loss(idx_flat, idx2_flat, table_pad, table_flat, lse):
    mesh = plsc.VectorSubcoreMesh(core_axis_name="c", subcore_axis_name="s")

    @functools.partial(
        pl.kernel,
        mesh=mesh,
        out_type=[
            jax.ShapeDtypeStruct((BT, VOCAB), jnp.float32),
            jax.ShapeDtypeStruct((NW, 16), jnp.float32),
        ],
        scratch_types=[
            pltpu.VMEM((PER_TILE,), jnp.int32),
            pltpu.VMEM((PER_TILE,), jnp.int32),
            pltpu.VMEM((CHUNK, VPAD), jnp.float32),
            pltpu.VMEM((CHUNK, VPAD), jnp.float32),
            pltpu.VMEM((CHUNK, VOCAB), jnp.float32),
            pltpu.VMEM((CHUNK, VOCAB), jnp.float32),
            pltpu.VMEM((PER_TILE,), jnp.float32),
            pltpu.VMEM((PER_TILE,), jnp.float32),
            pltpu.VMEM((16,), jnp.float32),
            pltpu.SemaphoreType.DMA,
            pltpu.SemaphoreType.DMA,
            pltpu.SemaphoreType.DMA,
            pltpu.SemaphoreType.DMA,
            pltpu.SemaphoreType.DMA,
        ],
    )
    def k(idx_hbm, idx2_hbm, table_hbm, tflat_hbm, lse_hbm,
          out_hbm, part_hbm,
          idx_v, idx2_v, pad0_v, pad1_v, rows0_v, rows1_v, tl_v, ls_v, acc_v,
          sem_g0, sem_g1, sem_s0, sem_s1, sem_e):
        pads = (pad0_v, pad1_v)
        rows = (rows0_v, rows1_v)
        sem_g = (sem_g0, sem_g1)
        sem_s = (sem_s0, sem_s1)
        wid = lax.axis_index("s") * NC + lax.axis_index("c")
        base = wid * PER_TILE
        pltpu.sync_copy(idx_hbm.at[pl.ds(base, PER_TILE)], idx_v)
        pltpu.sync_copy(idx2_hbm.at[pl.ds(base, PER_TILE)], idx2_v)

        def gather_rows(c, b, start):
            cp = pltpu.make_async_copy(
                table_hbm.at[idx_v.at[pl.ds(c * CHUNK, CHUNK)]],
                pads[b], sem_g[b],
            )
            if start:
                cp.start()
            return cp

        gather_rows(0, 0, True)
        gather_rows(1, 1, True)

        elem_cps = []
        for og in range(0, PER_TILE, LGRP):
            n = min(LGRP, PER_TILE - og)
            elem_cps.append(pltpu.async_copy(
                tflat_hbm.at[idx2_v.at[pl.ds(og, n)]],
                tl_v.at[pl.ds(og, n)], sem_e,
            ))
            elem_cps.append(pltpu.async_copy(
                lse_hbm.at[idx_v.at[pl.ds(og, n)]],
                ls_v.at[pl.ds(og, n)], sem_e,
            ))

        def pair_body(p, carry):
            for b in range(2):
                c = 2 * p + b
                o = c * CHUNK
                gather_rows(c, b, False).wait()
                @pl.when(c >= 2)
                def _():
                    pltpu.make_async_copy(
                        rows[b],
                        out_hbm.at[pl.ds(base + (c - 2) * CHUNK, CHUNK)],
                        sem_s[b],
                    ).wait()
                for r in range(CHUNK):
                    for j in range(61):
                        rows[b][r, pl.ds(16 * j, 16)] = (
                            pads[b][r, pl.ds(16 * j, 16)]
                        )
                    rows[b][r, pl.ds(984, 16)] = pads[b][r, pl.ds(984, 16)]
                    rows[b][r, pl.ds(976, 16)] = pads[b][r, pl.ds(976, 16)]
                pltpu.async_copy(
                    rows[b],
                    out_hbm.at[pl.ds(base + o, CHUNK)],
                    sem_s[b],
                )
                @pl.when(c + 2 < NCHUNK)
                def _():
                    gather_rows(c + 2, b, True)
            return carry

        lax.fori_loop(0, NCHUNK // 2, pair_body, 0)

        for b in range(2):
            pltpu.make_async_copy(
                rows[b],
                out_hbm.at[pl.ds(base + (NCHUNK - 2 + b) * CHUNK, CHUNK)],
                sem_s[b],
            ).wait()

        for cp in elem_cps:
            cp.wait()

        def loss_body(g, acc):
            og = g * 16
            return acc + (ls_v[pl.ds(og, 16)] - tl_v[pl.ds(og, 16)])

        acc = lax.fori_loop(
            0, PER_TILE // 16, loss_body, jnp.zeros((16,), jnp.float32)
        )
        acc_v[...] = acc * (1.0 / BT)
        pltpu.sync_copy(acc_v, part_hbm.at[wid])

    return k(idx_flat, idx2_flat, table_pad, table_flat, lse)


def kernel(index, targets, token_embedding_table):
    idx_flat = jnp.transpose(index).reshape(-1)
    tgt_flat = targets.reshape(-1)
    idx2_flat = idx_flat * VOCAB + tgt_flat
    lse = _row_logsumexp(token_embedding_table).reshape(VOCAB)
    table_pad = jnp.pad(token_embedding_table, ((0, 0), (0, VPAD - VOCAB)))
    logits, part = _sc_gather_and_loss(
        idx_flat, idx2_flat, table_pad,
        token_embedding_table.reshape(-1), lse
    )
    loss = jnp.sum(part)
    return (logits, loss)


# speedup vs baseline: 1.7055x; 1.2592x over previous
import functools

import jax
import jax.numpy as jnp
from jax import lax
from jax.experimental import pallas as pl
from jax.experimental.pallas import tpu as pltpu
from jax.experimental.pallas import tpu_sc as plsc

VOCAB = 1000
VPAD = 1024
BT = 51200
NC, NS = 2, 16
NW = NC * NS
PER_TILE = BT // NW
CHUNK = 40
NCHUNK = PER_TILE // CHUNK
LGRP = 128


def _row_logsumexp(table):
    def body(t_ref, o_ref):
        x = t_ref[...]
        m = jnp.max(x, axis=1, keepdims=True)
        s = jnp.sum(jnp.exp(x - m), axis=1, keepdims=True)
        o_ref[...] = jnp.log(s) + m

    return pl.pallas_call(
        body,
        out_shape=jax.ShapeDtypeStruct((VOCAB, 1), jnp.float32),
    )(table)


def _sc_gather_and_loss(idx_flat, idx2_flat, table_pad, table_flat, lse):
    mesh = plsc.VectorSubcoreMesh(core_axis_name="c", subcore_axis_name="s")

    @functools.partial(
        pl.kernel,
        mesh=mesh,
        out_type=[
            jax.ShapeDtypeStruct((BT, VPAD), jnp.float32),
            jax.ShapeDtypeStruct((NW, 16), jnp.float32),
        ],
        scratch_types=[
            pltpu.VMEM((PER_TILE,), jnp.int32),
            pltpu.VMEM((PER_TILE,), jnp.int32),
            pltpu.VMEM((CHUNK, VPAD), jnp.float32),
            pltpu.VMEM((CHUNK, VPAD), jnp.float32),
            pltpu.VMEM((PER_TILE,), jnp.float32),
            pltpu.VMEM((PER_TILE,), jnp.float32),
            pltpu.VMEM((16,), jnp.float32),
            pltpu.SemaphoreType.DMA,
            pltpu.SemaphoreType.DMA,
            pltpu.SemaphoreType.DMA,
            pltpu.SemaphoreType.DMA,
            pltpu.SemaphoreType.DMA,
        ],
    )
    def k(idx_hbm, idx2_hbm, table_hbm, tflat_hbm, lse_hbm,
          out_hbm, part_hbm,
          idx_v, idx2_v, pad0_v, pad1_v, tl_v, ls_v, acc_v,
          sem_g0, sem_g1, sem_s0, sem_s1, sem_e):
        pads = (pad0_v, pad1_v)
        sem_g = (sem_g0, sem_g1)
        sem_s = (sem_s0, sem_s1)
        wid = lax.axis_index("s") * NC + lax.axis_index("c")
        base = wid * PER_TILE
        pltpu.sync_copy(idx_hbm.at[pl.ds(base, PER_TILE)], idx_v)
        pltpu.sync_copy(idx2_hbm.at[pl.ds(base, PER_TILE)], idx2_v)

        def gather_rows(c, b, start):
            cp = pltpu.make_async_copy(
                table_hbm.at[idx_v.at[pl.ds(c * CHUNK, CHUNK)]],
                pads[b], sem_g[b],
            )
            if start:
                cp.start()
            return cp

        gather_rows(0, 0, True)
        gather_rows(1, 1, True)

        elem_cps = []
        for og in range(0, PER_TILE, LGRP):
            n = min(LGRP, PER_TILE - og)
            elem_cps.append(pltpu.async_copy(
                tflat_hbm.at[idx2_v.at[pl.ds(og, n)]],
                tl_v.at[pl.ds(og, n)], sem_e,
            ))
            elem_cps.append(pltpu.async_copy(
                lse_hbm.at[idx_v.at[pl.ds(og, n)]],
                ls_v.at[pl.ds(og, n)], sem_e,
            ))

        def pair_body(p, carry):
            for b in range(2):
                c = 2 * p + b
                o = c * CHUNK
                gather_rows(c, b, False).wait()
                pltpu.async_copy(
                    pads[b],
                    out_hbm.at[pl.ds(base + o, CHUNK)],
                    sem_s[b],
                ).wait()
                @pl.when(c + 2 < NCHUNK)
                def _():
                    gather_rows(c + 2, b, True)
            return carry

        lax.fori_loop(0, NCHUNK // 2, pair_body, 0)

        for cp in elem_cps:
            cp.wait()

        def loss_body(g, acc):
            og = g * 16
            return acc + (ls_v[pl.ds(og, 16)] - tl_v[pl.ds(og, 16)])

        acc = lax.fori_loop(
            0, PER_TILE // 16, loss_body, jnp.zeros((16,), jnp.float32)
        )
        acc_v[...] = acc * (1.0 / BT)
        pltpu.sync_copy(acc_v, part_hbm.at[wid])

    return k(idx_flat, idx2_flat, table_pad, table_flat, lse)


def kernel(index, targets, token_embedding_table):
    idx_flat = jnp.transpose(index).reshape(-1)
    tgt_flat = targets.reshape(-1)
    idx2_flat = idx_flat * VOCAB + tgt_flat
    lse = _row_logsumexp(token_embedding_table).reshape(VOCAB)
    table_pad = jnp.pad(token_embedding_table, ((0, 0), (0, VPAD - VOCAB)))
    logits_pad, part = _sc_gather_and_loss(
        idx_flat, idx2_flat, table_pad,
        token_embedding_table.reshape(-1), lse
    )
    loss = jnp.sum(part)
    return (logits_pad[:, :VOCAB], loss)
